# Initial kernel scaffold; baseline (speedup 1.0000x reference)
import jax
import jax.numpy as jnp
from jax.experimental import pallas as pl


def kernel(x, edge_index, batch, W_lin0, b_lin0, W_gam0, b_gam0, W_bet0, b_bet0, W_lin1, b_lin1, W_gam1, b_gam1, W_bet1, b_bet1, W_lin2, b_lin2, W_gam2, b_gam2, W_bet2, b_bet2):
    raise NotImplementedError("write your pallas kernel here")


# speedup vs baseline: 3.5450x; 3.5450x over previous
import functools

import jax
import jax.numpy as jnp
from jax import lax
from jax.experimental import pallas as pl
from jax.experimental.pallas import tpu as pltpu
from jax.experimental.pallas import tpu_sc as plsc

N = 10000
E = 160000
D = 256
G = 32
HALF = D // 2

ROW_BLOCK = 1000
NBLK = N // ROW_BLOCK

NUM_SUBCORES = 16
E_PER_TILE = E // NUM_SUBCORES
K = 80
NCHUNK = E_PER_TILE // K
ROWS_A = 624
ROWS_B = 640
LAST_BASE = 15 * ROWS_A
ZROWS = 208


def _three_matmuls(h, Wl, bl, Wg, bg, Wb, bb):
    dn = (((1,), (1,)), ((), ()))
    gamma = lax.dot_general(h, Wg, dn, preferred_element_type=jnp.float32) + bg
    beta = lax.dot_general(h, Wb, dn, preferred_element_type=jnp.float32) + bb
    xl = lax.dot_general(h, Wl, dn, preferred_element_type=jnp.float32) + bl
    return gamma, beta, xl


def _mm3_plain_body(h_ref, Wl_ref, bl_ref, Wg_ref, bg_ref, Wb_ref, bb_ref,
                    gamma_ref, beta_ref, xl0_ref, xl1_ref):
    gamma, beta, xl = _three_matmuls(h_ref[...], Wl_ref[...], bl_ref[...],
                                     Wg_ref[...], bg_ref[...],
                                     Wb_ref[...], bb_ref[...])
    gamma_ref[...] = gamma
    beta_ref[...] = beta
    xl0_ref[...] = xl[:, :HALF]
    xl1_ref[...] = xl[:, HALF:]


def _mm3_fused_body(g_ref, b_ref, a0_ref, a1_ref,
                    Wl_ref, bl_ref, Wg_ref, bg_ref, Wb_ref, bb_ref,
                    gamma_ref, beta_ref, xl0_ref, xl1_ref):
    agg = jnp.concatenate([a0_ref[...], a1_ref[...]], axis=1)
    h = jnp.maximum(g_ref[...] * agg + b_ref[...], 0.0)
    gamma, beta, xl = _three_matmuls(h, Wl_ref[...], bl_ref[...],
                                     Wg_ref[...], bg_ref[...],
                                     Wb_ref[...], bb_ref[...])
    gamma_ref[...] = gamma
    beta_ref[...] = beta
    xl0_ref[...] = xl[:, :HALF]
    xl1_ref[...] = xl[:, HALF:]


_W_SPEC = pl.BlockSpec((D, D), lambda i: (0, 0))
_B_SPEC = pl.BlockSpec((1, D), lambda i: (0, 0))
_FULL_SPEC = pl.BlockSpec((ROW_BLOCK, D), lambda i: (i, 0))
_HALF_SPEC = pl.BlockSpec((ROW_BLOCK, HALF), lambda i: (i, 0))

_MM_OUT_SHAPE = (
    jax.ShapeDtypeStruct((N, D), jnp.float32),
    jax.ShapeDtypeStruct((N, D), jnp.float32),
    jax.ShapeDtypeStruct((N, HALF), jnp.float32),
    jax.ShapeDtypeStruct((N, HALF), jnp.float32),
)
_MM_OUT_SPECS = (_FULL_SPEC, _FULL_SPEC, _HALF_SPEC, _HALF_SPEC)


def _mm3_plain(h, Wl, bl, Wg, bg, Wb, bb):
    return pl.pallas_call(
        _mm3_plain_body,
        grid=(NBLK,),
        in_specs=[_FULL_SPEC, _W_SPEC, _B_SPEC, _W_SPEC, _B_SPEC, _W_SPEC,
                  _B_SPEC],
        out_specs=_MM_OUT_SPECS,
        out_shape=_MM_OUT_SHAPE,
    )(h, Wl, bl.reshape(1, D), Wg, bg.reshape(1, D), Wb, bb.reshape(1, D))


def _mm3_fused(gamma, beta, a0, a1, Wl, bl, Wg, bg, Wb, bb):
    return pl.pallas_call(
        _mm3_fused_body,
        grid=(NBLK,),
        in_specs=[_FULL_SPEC, _FULL_SPEC, _HALF_SPEC, _HALF_SPEC,
                  _W_SPEC, _B_SPEC, _W_SPEC, _B_SPEC, _W_SPEC, _B_SPEC],
        out_specs=_MM_OUT_SPECS,
        out_shape=_MM_OUT_SHAPE,
    )(gamma, beta, a0, a1,
      Wl, bl.reshape(1, D), Wg, bg.reshape(1, D), Wb, bb.reshape(1, D))


def _pool_body(g_ref, b_ref, a0_ref, a1_ref, batch_ref, out_ref,
               sums_ref, counts_ref):
    i = pl.program_id(0)
    agg = jnp.concatenate([a0_ref[...], a1_ref[...]], axis=1)
    h = jnp.maximum(g_ref[...] * agg + b_ref[...], 0.0)
    b = batch_ref[0, 0, :]
    seg = lax.broadcasted_iota(jnp.int32, (G, ROW_BLOCK), 0)
    onehot = (b[None, :] == seg).astype(jnp.float32)
    psum = lax.dot_general(onehot, h, (((1,), (0,)), ((), ())),
                           preferred_element_type=jnp.float32)
    pcnt = jnp.broadcast_to(jnp.sum(onehot, axis=1)[:, None], (G, D))

    @pl.when(i == 0)
    def _():
        sums_ref[...] = jnp.zeros_like(sums_ref)
        counts_ref[...] = jnp.zeros_like(counts_ref)

    sums_ref[...] += psum
    counts_ref[...] += pcnt

    @pl.when(i == NBLK - 1)
    def _():
        out_ref[...] = sums_ref[...] / jnp.maximum(counts_ref[...], 1.0)


def _pool(gamma, beta, a0, a1, batch3):
    return pl.pallas_call(
        _pool_body,
        grid=(NBLK,),
        in_specs=[_FULL_SPEC, _FULL_SPEC, _HALF_SPEC, _HALF_SPEC,
                  pl.BlockSpec((1, 1, ROW_BLOCK), lambda i: (i, 0, 0))],
        out_specs=pl.BlockSpec((G, D), lambda i: (0, 0)),
        out_shape=jax.ShapeDtypeStruct((G, D), jnp.float32),
        scratch_shapes=[pltpu.VMEM((G, D), jnp.float32),
                        pltpu.VMEM((G, D), jnp.float32)],
    )(gamma, beta, a0, a1, batch3)


def _edge_agg(xl0, xl1, src, dst):
    mesh = plsc.VectorSubcoreMesh(core_axis_name="c", subcore_axis_name="s")

    @functools.partial(
        pl.kernel,
        mesh=mesh,
        out_type=(jax.ShapeDtypeStruct((N, HALF), jnp.float32),
                  jax.ShapeDtypeStruct((N, HALF), jnp.float32)),
        scratch_types=[
            pltpu.VMEM((K,), jnp.int32),
            pltpu.VMEM((K,), jnp.int32),
            pltpu.VMEM((K, HALF), jnp.float32),
            pltpu.VMEM((ZROWS, HALF), jnp.float32),
            pltpu.VMEM_SHARED((N, HALF), jnp.float32),
            pltpu.SemaphoreType.DMA,
        ],
    )
    def kern(xl0_hbm, xl1_hbm, src_hbm, dst_hbm, agg0_hbm, agg1_hbm,
             srcv, dstv, rows, zbuf, acc, sem):
        c = lax.axis_index("c")
        s = lax.axis_index("s")

        zero16 = jnp.zeros((16,), jnp.float32)

        def zfill(i, carry):
            zbuf[i // 8, pl.ds((i % 8) * 16, 16)] = zero16
            return carry

        lax.fori_loop(0, ZROWS * 8, zfill, 0)
        for kk in range(ROWS_A // ZROWS):
            pltpu.sync_copy(zbuf, acc.at[pl.ds(s * ROWS_A + kk * ZROWS, ZROWS)])

        @pl.when(s == NUM_SUBCORES - 1)
        def _():
            pltpu.sync_copy(zbuf.at[pl.ds(0, 16)],
                            acc.at[pl.ds(LAST_BASE + ROWS_A, 16)])

        plsc.subcore_barrier()

        def run(xl_hbm, out_hbm):
            base = s * E_PER_TILE

            def body(i, carry):
                off = base + i * K
                pltpu.sync_copy(src_hbm.at[pl.ds(off, K)], srcv)
                pltpu.sync_copy(dst_hbm.at[pl.ds(off, K)], dstv)
                pltpu.async_copy(xl_hbm.at[srcv], rows, sem).wait()
                pltpu.sync_copy(rows, acc.at[dstv], add=True)
                return carry

            lax.fori_loop(0, NCHUNK, body, 0)
            plsc.subcore_barrier()

            @pl.when(s < NUM_SUBCORES - 1)
            def _():
                pltpu.sync_copy(acc.at[pl.ds(s * ROWS_A, ROWS_A)],
                                out_hbm.at[pl.ds(s * ROWS_A, ROWS_A)])

            @pl.when(s == NUM_SUBCORES - 1)
            def _():
                pltpu.sync_copy(acc.at[pl.ds(LAST_BASE, ROWS_B)],
                                out_hbm.at[pl.ds(LAST_BASE, ROWS_B)])

        @pl.when(c == 0)
        def _():
            run(xl0_hbm, agg0_hbm)

        @pl.when(c == 1)
        def _():
            run(xl1_hbm, agg1_hbm)

    return kern(xl0, xl1, src, dst)


def kernel(x, edge_index, batch,
           W_lin0, b_lin0, W_gam0, b_gam0, W_bet0, b_bet0,
           W_lin1, b_lin1, W_gam1, b_gam1, W_bet1, b_bet1,
           W_lin2, b_lin2, W_gam2, b_gam2, W_bet2, b_bet2):
    src = edge_index[0]
    dst = edge_index[1]
    batch3 = batch.reshape(NBLK, 1, ROW_BLOCK)

    gamma, beta, xl0, xl1 = _mm3_plain(x, W_lin0, b_lin0, W_gam0, b_gam0,
                                       W_bet0, b_bet0)
    a0, a1 = _edge_agg(xl0, xl1, src, dst)

    gamma, beta, xl0, xl1 = _mm3_fused(gamma, beta, a0, a1,
                                       W_lin1, b_lin1, W_gam1, b_gam1,
                                       W_bet1, b_bet1)
    a0, a1 = _edge_agg(xl0, xl1, src, dst)

    gamma, beta, xl0, xl1 = _mm3_fused(gamma, beta, a0, a1,
                                       W_lin2, b_lin2, W_gam2, b_gam2,
                                       W_bet2, b_bet2)
    a0, a1 = _edge_agg(xl0, xl1, src, dst)

    return _pool(gamma, beta, a0, a1, batch3)


# speedup vs baseline: 8.1349x; 2.2947x over previous
import functools

import jax
import jax.numpy as jnp
from jax import lax
from jax.experimental import pallas as pl
from jax.experimental.pallas import tpu as pltpu
from jax.experimental.pallas import tpu_sc as plsc

N = 10000
E = 160000
D = 256
G = 32
HALF = D // 2

ROW_BLOCK = 1000
NBLK = N // ROW_BLOCK

NUM_SUBCORES = 16
E_PER_TILE = E // NUM_SUBCORES
K = 80
NCHUNK = E_PER_TILE // K
ROWS_A = 624
ROWS_B = 640
LAST_BASE = 15 * ROWS_A
ZROWS = 16


def _three_matmuls(h, Wl, bl, Wg, bg, Wb, bb):
    dn = (((1,), (1,)), ((), ()))
    gamma = lax.dot_general(h, Wg, dn, preferred_element_type=jnp.float32) + bg
    beta = lax.dot_general(h, Wb, dn, preferred_element_type=jnp.float32) + bb
    xl = lax.dot_general(h, Wl, dn, preferred_element_type=jnp.float32) + bl
    return gamma, beta, xl


def _mm3_plain_body(h_ref, Wl_ref, bl_ref, Wg_ref, bg_ref, Wb_ref, bb_ref,
                    gamma_ref, beta_ref, xl0_ref, xl1_ref):
    gamma, beta, xl = _three_matmuls(h_ref[...], Wl_ref[...], bl_ref[...],
                                     Wg_ref[...], bg_ref[...],
                                     Wb_ref[...], bb_ref[...])
    gamma_ref[...] = gamma
    beta_ref[...] = beta
    xl0_ref[...] = xl[:, :HALF]
    xl1_ref[...] = xl[:, HALF:]


def _mm3_fused_body(g_ref, b_ref, a0_ref, a1_ref,
                    Wl_ref, bl_ref, Wg_ref, bg_ref, Wb_ref, bb_ref,
                    gamma_ref, beta_ref, xl0_ref, xl1_ref):
    agg = jnp.concatenate([a0_ref[...], a1_ref[...]], axis=1)
    h = jnp.maximum(g_ref[...] * agg + b_ref[...], 0.0)
    gamma, beta, xl = _three_matmuls(h, Wl_ref[...], bl_ref[...],
                                     Wg_ref[...], bg_ref[...],
                                     Wb_ref[...], bb_ref[...])
    gamma_ref[...] = gamma
    beta_ref[...] = beta
    xl0_ref[...] = xl[:, :HALF]
    xl1_ref[...] = xl[:, HALF:]


_W_SPEC = pl.BlockSpec((D, D), lambda i: (0, 0))
_B_SPEC = pl.BlockSpec((1, D), lambda i: (0, 0))
_FULL_SPEC = pl.BlockSpec((ROW_BLOCK, D), lambda i: (i, 0))
_HALF_SPEC = pl.BlockSpec((ROW_BLOCK, HALF), lambda i: (i, 0))

_MM_OUT_SHAPE = (
    jax.ShapeDtypeStruct((N, D), jnp.float32),
    jax.ShapeDtypeStruct((N, D), jnp.float32),
    jax.ShapeDtypeStruct((N, HALF), jnp.float32),
    jax.ShapeDtypeStruct((N, HALF), jnp.float32),
)
_MM_OUT_SPECS = (_FULL_SPEC, _FULL_SPEC, _HALF_SPEC, _HALF_SPEC)


def _mm3_plain(h, Wl, bl, Wg, bg, Wb, bb):
    return pl.pallas_call(
        _mm3_plain_body,
        grid=(NBLK,),
        in_specs=[_FULL_SPEC, _W_SPEC, _B_SPEC, _W_SPEC, _B_SPEC, _W_SPEC,
                  _B_SPEC],
        out_specs=_MM_OUT_SPECS,
        out_shape=_MM_OUT_SHAPE,
    )(h, Wl, bl.reshape(1, D), Wg, bg.reshape(1, D), Wb, bb.reshape(1, D))


def _mm3_fused(gamma, beta, a0, a1, Wl, bl, Wg, bg, Wb, bb):
    return pl.pallas_call(
        _mm3_fused_body,
        grid=(NBLK,),
        in_specs=[_FULL_SPEC, _FULL_SPEC, _HALF_SPEC, _HALF_SPEC,
                  _W_SPEC, _B_SPEC, _W_SPEC, _B_SPEC, _W_SPEC, _B_SPEC],
        out_specs=_MM_OUT_SPECS,
        out_shape=_MM_OUT_SHAPE,
    )(gamma, beta, a0, a1,
      Wl, bl.reshape(1, D), Wg, bg.reshape(1, D), Wb, bb.reshape(1, D))


def _pool_body(g_ref, b_ref, a0_ref, a1_ref, batch_ref, out_ref,
               sums_ref, counts_ref):
    i = pl.program_id(0)
    agg = jnp.concatenate([a0_ref[...], a1_ref[...]], axis=1)
    h = jnp.maximum(g_ref[...] * agg + b_ref[...], 0.0)
    b = batch_ref[0, 0, :]
    seg = lax.broadcasted_iota(jnp.int32, (G, ROW_BLOCK), 0)
    onehot = (b[None, :] == seg).astype(jnp.float32)
    psum = lax.dot_general(onehot, h, (((1,), (0,)), ((), ())),
                           preferred_element_type=jnp.float32)
    pcnt = jnp.broadcast_to(jnp.sum(onehot, axis=1)[:, None], (G, D))

    @pl.when(i == 0)
    def _():
        sums_ref[...] = jnp.zeros_like(sums_ref)
        counts_ref[...] = jnp.zeros_like(counts_ref)

    sums_ref[...] += psum
    counts_ref[...] += pcnt

    @pl.when(i == NBLK - 1)
    def _():
        out_ref[...] = sums_ref[...] / jnp.maximum(counts_ref[...], 1.0)


def _pool(gamma, beta, a0, a1, batch3):
    return pl.pallas_call(
        _pool_body,
        grid=(NBLK,),
        in_specs=[_FULL_SPEC, _FULL_SPEC, _HALF_SPEC, _HALF_SPEC,
                  pl.BlockSpec((1, 1, ROW_BLOCK), lambda i: (i, 0, 0))],
        out_specs=pl.BlockSpec((G, D), lambda i: (0, 0)),
        out_shape=jax.ShapeDtypeStruct((G, D), jnp.float32),
        scratch_shapes=[pltpu.VMEM((G, D), jnp.float32),
                        pltpu.VMEM((G, D), jnp.float32)],
    )(gamma, beta, a0, a1, batch3)


NBUF = 2


def _edge_agg(xl0, xl1, packed3):
    mesh = plsc.VectorSubcoreMesh(core_axis_name="c", subcore_axis_name="s")

    @functools.partial(
        pl.kernel,
        mesh=mesh,
        out_type=(jax.ShapeDtypeStruct((N, HALF), jnp.float32),
                  jax.ShapeDtypeStruct((N, HALF), jnp.float32)),
        scratch_types=[
            pltpu.VMEM((NCHUNK, K), jnp.int32),
            pltpu.VMEM((NBUF, K), jnp.int32),
            pltpu.VMEM((NBUF, K), jnp.int32),
            pltpu.VMEM((NBUF, K, HALF), jnp.float32),
            pltpu.VMEM((ZROWS, HALF), jnp.float32),
            pltpu.VMEM_SHARED((N, HALF), jnp.float32),
            pltpu.SemaphoreType.DMA,
        ],
    )
    def kern(xl0_hbm, xl1_hbm, packed_hbm, agg0_hbm, agg1_hbm,
             packed_all, srcb, dstb, rows, zbuf, acc, sem):
        c = lax.axis_index("c")
        s = lax.axis_index("s")

        zero16 = jnp.zeros((16,), jnp.float32)

        def zfill(i, carry):
            zbuf[i // 8, pl.ds((i % 8) * 16, 16)] = zero16
            return carry

        lax.fori_loop(0, ZROWS * 8, zfill, 0)

        def zcopy(kk, carry):
            pltpu.sync_copy(zbuf, acc.at[pl.ds(s * ROWS_A + kk * ZROWS, ZROWS)])
            return carry

        lax.fori_loop(0, ROWS_A // ZROWS, zcopy, 0)

        @pl.when(s == NUM_SUBCORES - 1)
        def _():
            pltpu.sync_copy(zbuf, acc.at[pl.ds(LAST_BASE + ROWS_A, ZROWS)])

        plsc.subcore_barrier()

        def run(xl_hbm, out_hbm):
            pltpu.sync_copy(packed_hbm.at[s], packed_all)

            def unpack(c, b):
                for j in range(K // 16):
                    p = packed_all[c, pl.ds(j * 16, 16)]
                    srcb[b, pl.ds(j * 16, 16)] = p & 0xFFFF
                    dstb[b, pl.ds(j * 16, 16)] = lax.shift_right_logical(
                        p, 16)

            def fire(c, b):
                unpack(c, b)
                pltpu.async_copy(xl_hbm.at[srcb.at[b]], rows.at[b], sem)

            def drain(b):
                pltpu.make_async_copy(xl_hbm.at[pl.ds(0, K)], rows.at[b],
                                      sem).wait()

            def scatter(b):
                pltpu.sync_copy(rows.at[b], acc.at[dstb.at[b]], add=True)

            for b in range(NBUF):
                fire(b, b)

            def body(u, carry):
                for b in range(NBUF):
                    c = u * NBUF + b
                    drain(b)
                    scatter(b)
                    fire(c + NBUF, b)
                return carry

            lax.fori_loop(0, (NCHUNK - 3) // NBUF, body, 0)
            drain(0)
            scatter(0)
            fire(NCHUNK - 1, 0)
            drain(1)
            scatter(1)
            drain(0)
            scatter(0)
            plsc.subcore_barrier()

            @pl.when(s < NUM_SUBCORES - 1)
            def _():
                pltpu.sync_copy(acc.at[pl.ds(s * ROWS_A, ROWS_A)],
                                out_hbm.at[pl.ds(s * ROWS_A, ROWS_A)])

            @pl.when(s == NUM_SUBCORES - 1)
            def _():
                pltpu.sync_copy(acc.at[pl.ds(LAST_BASE, ROWS_B)],
                                out_hbm.at[pl.ds(LAST_BASE, ROWS_B)])

        @pl.when(c == 0)
        def _():
            run(xl0_hbm, agg0_hbm)

        @pl.when(c == 1)
        def _():
            run(xl1_hbm, agg1_hbm)

    return kern(xl0, xl1, packed3)


def kernel(x, edge_index, batch,
           W_lin0, b_lin0, W_gam0, b_gam0, W_bet0, b_bet0,
           W_lin1, b_lin1, W_gam1, b_gam1, W_bet1, b_bet1,
           W_lin2, b_lin2, W_gam2, b_gam2, W_bet2, b_bet2):
    packed3 = ((edge_index[1] << 16) | edge_index[0]).reshape(
        NUM_SUBCORES, NCHUNK, K)
    batch3 = batch.reshape(NBLK, 1, ROW_BLOCK)

    gamma, beta, xl0, xl1 = _mm3_plain(x, W_lin0, b_lin0, W_gam0, b_gam0,
                                       W_bet0, b_bet0)
    a0, a1 = _edge_agg(xl0, xl1, packed3)

    gamma, beta, xl0, xl1 = _mm3_fused(gamma, beta, a0, a1,
                                       W_lin1, b_lin1, W_gam1, b_gam1,
                                       W_bet1, b_bet1)
    a0, a1 = _edge_agg(xl0, xl1, packed3)

    gamma, beta, xl0, xl1 = _mm3_fused(gamma, beta, a0, a1,
                                       W_lin2, b_lin2, W_gam2, b_gam2,
                                       W_bet2, b_bet2)
    a0, a1 = _edge_agg(xl0, xl1, packed3)

    return _pool(gamma, beta, a0, a1, batch3)


# speedup vs baseline: 8.1462x; 1.0014x over previous
import functools

import jax
import jax.numpy as jnp
from jax import lax
from jax.experimental import pallas as pl
from jax.experimental.pallas import tpu as pltpu
from jax.experimental.pallas import tpu_sc as plsc

N = 10000
E = 160000
D = 256
G = 32
HALF = D // 2

ROW_BLOCK = 1000
NBLK = N // ROW_BLOCK

NUM_SUBCORES = 16
E_PER_TILE = E // NUM_SUBCORES
K = 80
NCHUNK = E_PER_TILE // K
ROWS_A = 624
ROWS_B = 640
LAST_BASE = 15 * ROWS_A
ZROWS = 32


def _three_matmuls(h, Wl, bl, Wg, bg, Wb, bb):
    dn = (((1,), (1,)), ((), ()))
    gamma = lax.dot_general(h, Wg, dn, preferred_element_type=jnp.float32) + bg
    beta = lax.dot_general(h, Wb, dn, preferred_element_type=jnp.float32) + bb
    xl = lax.dot_general(h, Wl, dn, preferred_element_type=jnp.float32) + bl
    return gamma, beta, xl


def _mm3_plain_body(h_ref, Wl_ref, bl_ref, Wg_ref, bg_ref, Wb_ref, bb_ref,
                    gamma_ref, beta_ref, xl0_ref, xl1_ref):
    gamma, beta, xl = _three_matmuls(h_ref[...], Wl_ref[...], bl_ref[...],
                                     Wg_ref[...], bg_ref[...],
                                     Wb_ref[...], bb_ref[...])
    gamma_ref[...] = gamma
    beta_ref[...] = beta
    xl0_ref[...] = xl[:, :HALF]
    xl1_ref[...] = xl[:, HALF:]


def _mm3_fused_body(g_ref, b_ref, a0_ref, a1_ref,
                    Wl_ref, bl_ref, Wg_ref, bg_ref, Wb_ref, bb_ref,
                    gamma_ref, beta_ref, xl0_ref, xl1_ref):
    agg = jnp.concatenate([a0_ref[...], a1_ref[...]], axis=1)
    h = jnp.maximum(g_ref[...] * agg + b_ref[...], 0.0)
    gamma, beta, xl = _three_matmuls(h, Wl_ref[...], bl_ref[...],
                                     Wg_ref[...], bg_ref[...],
                                     Wb_ref[...], bb_ref[...])
    gamma_ref[...] = gamma
    beta_ref[...] = beta
    xl0_ref[...] = xl[:, :HALF]
    xl1_ref[...] = xl[:, HALF:]


_W_SPEC = pl.BlockSpec((D, D), lambda i: (0, 0))
_B_SPEC = pl.BlockSpec((1, D), lambda i: (0, 0))
_FULL_SPEC = pl.BlockSpec((ROW_BLOCK, D), lambda i: (i, 0))
_HALF_SPEC = pl.BlockSpec((ROW_BLOCK, HALF), lambda i: (i, 0))

_MM_OUT_SHAPE = (
    jax.ShapeDtypeStruct((N, D), jnp.float32),
    jax.ShapeDtypeStruct((N, D), jnp.float32),
    jax.ShapeDtypeStruct((N, HALF), jnp.float32),
    jax.ShapeDtypeStruct((N, HALF), jnp.float32),
)
_MM_OUT_SPECS = (_FULL_SPEC, _FULL_SPEC, _HALF_SPEC, _HALF_SPEC)


def _mm3_plain(h, Wl, bl, Wg, bg, Wb, bb):
    return pl.pallas_call(
        _mm3_plain_body,
        grid=(NBLK,),
        in_specs=[_FULL_SPEC, _W_SPEC, _B_SPEC, _W_SPEC, _B_SPEC, _W_SPEC,
                  _B_SPEC],
        out_specs=_MM_OUT_SPECS,
        out_shape=_MM_OUT_SHAPE,
    )(h, Wl, bl.reshape(1, D), Wg, bg.reshape(1, D), Wb, bb.reshape(1, D))


def _mm3_fused(gamma, beta, a0, a1, Wl, bl, Wg, bg, Wb, bb):
    return pl.pallas_call(
        _mm3_fused_body,
        grid=(NBLK,),
        in_specs=[_FULL_SPEC, _FULL_SPEC, _HALF_SPEC, _HALF_SPEC,
                  _W_SPEC, _B_SPEC, _W_SPEC, _B_SPEC, _W_SPEC, _B_SPEC],
        out_specs=_MM_OUT_SPECS,
        out_shape=_MM_OUT_SHAPE,
    )(gamma, beta, a0, a1,
      Wl, bl.reshape(1, D), Wg, bg.reshape(1, D), Wb, bb.reshape(1, D))


def _pool_body(g_ref, b_ref, a0_ref, a1_ref, batch_ref, out_ref,
               sums_ref, counts_ref):
    i = pl.program_id(0)
    agg = jnp.concatenate([a0_ref[...], a1_ref[...]], axis=1)
    h = jnp.maximum(g_ref[...] * agg + b_ref[...], 0.0)
    b = batch_ref[0, 0, :]
    seg = lax.broadcasted_iota(jnp.int32, (G, ROW_BLOCK), 0)
    onehot = (b[None, :] == seg).astype(jnp.float32)
    psum = lax.dot_general(onehot, h, (((1,), (0,)), ((), ())),
                           preferred_element_type=jnp.float32)
    pcnt = jnp.broadcast_to(jnp.sum(onehot, axis=1)[:, None], (G, D))

    @pl.when(i == 0)
    def _():
        sums_ref[...] = jnp.zeros_like(sums_ref)
        counts_ref[...] = jnp.zeros_like(counts_ref)

    sums_ref[...] += psum
    counts_ref[...] += pcnt

    @pl.when(i == NBLK - 1)
    def _():
        out_ref[...] = sums_ref[...] / jnp.maximum(counts_ref[...], 1.0)


def _pool(gamma, beta, a0, a1, batch3):
    return pl.pallas_call(
        _pool_body,
        grid=(NBLK,),
        in_specs=[_FULL_SPEC, _FULL_SPEC, _HALF_SPEC, _HALF_SPEC,
                  pl.BlockSpec((1, 1, ROW_BLOCK), lambda i: (i, 0, 0))],
        out_specs=pl.BlockSpec((G, D), lambda i: (0, 0)),
        out_shape=jax.ShapeDtypeStruct((G, D), jnp.float32),
        scratch_shapes=[pltpu.VMEM((G, D), jnp.float32),
                        pltpu.VMEM((G, D), jnp.float32)],
    )(gamma, beta, a0, a1, batch3)


NBUF = 2


def _edge_agg(xl0, xl1, packed3):
    mesh = plsc.VectorSubcoreMesh(core_axis_name="c", subcore_axis_name="s")

    @functools.partial(
        pl.kernel,
        mesh=mesh,
        out_type=(jax.ShapeDtypeStruct((N, HALF), jnp.float32),
                  jax.ShapeDtypeStruct((N, HALF), jnp.float32)),
        scratch_types=[
            pltpu.VMEM((NCHUNK, K), jnp.int32),
            pltpu.VMEM((NBUF, K), jnp.int32),
            pltpu.VMEM((NBUF, K), jnp.int32),
            pltpu.VMEM((NBUF, K, HALF), jnp.float32),
            pltpu.VMEM((ZROWS, HALF), jnp.float32),
            pltpu.VMEM_SHARED((N, HALF), jnp.float32),
            pltpu.SemaphoreType.DMA,
        ],
    )
    def kern(xl0_hbm, xl1_hbm, packed_hbm, agg0_hbm, agg1_hbm,
             packed_all, srcb, dstb, rows, zbuf, acc, sem):
        c = lax.axis_index("c")
        s = lax.axis_index("s")

        pltpu.sync_copy(packed_hbm.at[s], packed_all)

        zero16 = jnp.zeros((16,), jnp.float32)

        def zfill(i, carry):
            zbuf[i // 8, pl.ds((i % 8) * 16, 16)] = zero16
            return carry

        lax.fori_loop(0, ZROWS * 8, zfill, 0)

        def zcopy(kk, carry):
            pltpu.sync_copy(zbuf, acc.at[pl.ds(s * ROWS_A + kk * ZROWS, ZROWS)])
            return carry

        lax.fori_loop(0, ROWS_A // ZROWS, zcopy, 0)

        @pl.when(s < NUM_SUBCORES - 1)
        def _():
            pltpu.sync_copy(
                zbuf.at[pl.ds(0, 16)],
                acc.at[pl.ds(s * ROWS_A + (ROWS_A // ZROWS) * ZROWS, 16)])

        @pl.when(s == NUM_SUBCORES - 1)
        def _():
            pltpu.sync_copy(
                zbuf, acc.at[pl.ds(LAST_BASE + (ROWS_A // ZROWS) * ZROWS,
                                   ZROWS)])

        plsc.subcore_barrier()

        def run(xl_hbm, out_hbm):
            def unpack(c, b):
                for j in range(K // 16):
                    p = packed_all[c, pl.ds(j * 16, 16)]
                    srcb[b, pl.ds(j * 16, 16)] = p & 0xFFFF
                    dstb[b, pl.ds(j * 16, 16)] = lax.shift_right_logical(
                        p, 16)

            def fire(c, b):
                unpack(c, b)
                pltpu.async_copy(xl_hbm.at[srcb.at[b]], rows.at[b], sem)

            def drain(b):
                pltpu.make_async_copy(xl_hbm.at[pl.ds(0, K)], rows.at[b],
                                      sem).wait()

            def scatter(b):
                pltpu.sync_copy(rows.at[b], acc.at[dstb.at[b]], add=True)

            for b in range(NBUF):
                fire(b, b)

            def body(u, carry):
                for b in range(NBUF):
                    c = u * NBUF + b
                    drain(b)
                    scatter(b)
                    fire(c + NBUF, b)
                return carry

            lax.fori_loop(0, (NCHUNK - 3) // NBUF, body, 0)
            drain(0)
            scatter(0)
            fire(NCHUNK - 1, 0)
            drain(1)
            scatter(1)
            drain(0)
            scatter(0)
            plsc.subcore_barrier()

            @pl.when(s < NUM_SUBCORES - 1)
            def _():
                pltpu.sync_copy(acc.at[pl.ds(s * ROWS_A, ROWS_A)],
                                out_hbm.at[pl.ds(s * ROWS_A, ROWS_A)])

            @pl.when(s == NUM_SUBCORES - 1)
            def _():
                pltpu.sync_copy(acc.at[pl.ds(LAST_BASE, ROWS_B)],
                                out_hbm.at[pl.ds(LAST_BASE, ROWS_B)])

        @pl.when(c == 0)
        def _():
            run(xl0_hbm, agg0_hbm)

        @pl.when(c == 1)
        def _():
            run(xl1_hbm, agg1_hbm)

    return kern(xl0, xl1, packed3)


def kernel(x, edge_index, batch,
           W_lin0, b_lin0, W_gam0, b_gam0, W_bet0, b_bet0,
           W_lin1, b_lin1, W_gam1, b_gam1, W_bet1, b_bet1,
           W_lin2, b_lin2, W_gam2, b_gam2, W_bet2, b_bet2):
    packed3 = ((edge_index[1] << 16) | edge_index[0]).reshape(
        NUM_SUBCORES, NCHUNK, K)
    batch3 = batch.reshape(NBLK, 1, ROW_BLOCK)

    gamma, beta, xl0, xl1 = _mm3_plain(x, W_lin0, b_lin0, W_gam0, b_gam0,
                                       W_bet0, b_bet0)
    a0, a1 = _edge_agg(xl0, xl1, packed3)

    gamma, beta, xl0, xl1 = _mm3_fused(gamma, beta, a0, a1,
                                       W_lin1, b_lin1, W_gam1, b_gam1,
                                       W_bet1, b_bet1)
    a0, a1 = _edge_agg(xl0, xl1, packed3)

    gamma, beta, xl0, xl1 = _mm3_fused(gamma, beta, a0, a1,
                                       W_lin2, b_lin2, W_gam2, b_gam2,
                                       W_bet2, b_bet2)
    a0, a1 = _edge_agg(xl0, xl1, packed3)

    return _pool(gamma, beta, a0, a1, batch3)
